# baseline (speedup 1.0000x reference)
import functools
import math

import jax
import jax.numpy as jnp
from jax import lax
from jax.experimental import pallas as pl
from jax.experimental.pallas import tpu as pltpu
from jax.experimental.pallas import tpu_sc as plsc

SMOOTH = 0.1
VOCAB = 32000
FILL = SMOOTH / (VOCAB - 1)
CONF = 1.0 - SMOOTH
C_CONST = SMOOTH * math.log(FILL) + CONF * math.log(CONF)

ROWS_PER_BLOCK = 128
N_WORKERS = 32
LANES = 16


def _stream_kernel(pred_lo_ref, pred_hi_ref, tgt_ref, num_ref, acc_ref, *,
                   num_blocks, half):
    i = pl.program_id(0)

    tgt = tgt_ref[...]
    lo = pred_lo_ref[...]
    hi = pred_hi_ref[...]
    col = jax.lax.broadcasted_iota(jnp.int32, lo.shape, 1)
    w_lo = jnp.where(col == tgt, CONF, FILL)
    w_hi = jnp.where(col + half == tgt, CONF, FILL)
    wsum = (jnp.sum(w_lo * lo, axis=1, keepdims=True)
            + jnp.sum(w_hi * hi, axis=1, keepdims=True))
    mask = (tgt != 0).astype(jnp.float32)
    block_loss = jnp.sum(mask * (C_CONST - wsum))

    @pl.when(i == 0)
    def _():
        acc_ref[0, 0] = 0.0

    acc_ref[0, 0] += block_loss

    @pl.when(i == num_blocks - 1)
    def _():
        num_ref[...] = jnp.full((1, 1), acc_ref[0, 0], dtype=jnp.float32)


def _sc_count(tgt_hbm, out_hbm, tgt_v, acc_v):
    wid = lax.axis_index("s") * 2 + lax.axis_index("c")
    per_w = 2048 // N_WORKERS
    base = wid * per_w
    pltpu.sync_copy(tgt_hbm.at[pl.ds(base, per_w)], tgt_v)
    acc = jnp.zeros((LANES,), jnp.float32)
    for j in range(per_w // LANES):
        tv = tgt_v[pl.ds(j * LANES, LANES)]
        acc = acc + jnp.where(tv != 0, 1.0, 0.0)
    acc_v[...] = acc
    pltpu.sync_copy(acc_v, out_hbm.at[wid])


def _combine_kernel(num_ref, cp_ref, out_ref):
    cnt = jnp.sum(cp_ref[...])
    out_ref[...] = jnp.full((1, 1), num_ref[0, 0] / cnt, dtype=jnp.float32)


def kernel(prediction, target):
    _, n_tok, vocab = prediction.shape
    pred2d = prediction.reshape(n_tok, vocab)
    tgt_col = target.reshape(n_tok, 1)
    tgt_flat = target.reshape(n_tok)
    num_blocks = n_tok // ROWS_PER_BLOCK
    half = vocab // 2

    sc_call = pl.kernel(
        _sc_count,
        out_type=jax.ShapeDtypeStruct((N_WORKERS, LANES), jnp.float32),
        mesh=plsc.VectorSubcoreMesh(core_axis_name="c", subcore_axis_name="s"),
        scratch_types=[
            pltpu.VMEM((n_tok // N_WORKERS,), jnp.int32),
            pltpu.VMEM((LANES,), jnp.float32),
        ],
    )
    c_part = sc_call(tgt_flat)

    num = pl.pallas_call(
        functools.partial(_stream_kernel, num_blocks=num_blocks, half=half),
        grid=(num_blocks,),
        in_specs=[
            pl.BlockSpec((ROWS_PER_BLOCK, half), lambda i: (i, 0)),
            pl.BlockSpec((ROWS_PER_BLOCK, half), lambda i: (i, 1)),
            pl.BlockSpec((ROWS_PER_BLOCK, 1), lambda i: (i, 0)),
        ],
        out_specs=pl.BlockSpec((1, 1), lambda i: (0, 0)),
        out_shape=jax.ShapeDtypeStruct((1, 1), jnp.float32),
        scratch_shapes=[pltpu.SMEM((1, 1), jnp.float32)],
    )(pred2d, pred2d, tgt_col)

    out = pl.pallas_call(
        _combine_kernel,
        grid=(1,),
        in_specs=[
            pl.BlockSpec((1, 1), lambda i: (0, 0)),
            pl.BlockSpec((N_WORKERS, LANES), lambda i: (0, 0)),
        ],
        out_specs=pl.BlockSpec((1, 1), lambda i: (0, 0)),
        out_shape=jax.ShapeDtypeStruct((1, 1), jnp.float32),
    )(num, c_part)
    return out[0, 0]


# speedup vs baseline: 1.2178x; 1.2178x over previous
import functools

import jax
import jax.numpy as jnp
from jax.experimental import pallas as pl
from jax.experimental.pallas import tpu as pltpu

SMOOTH = 0.1
VOCAB = 32000
FILL = SMOOTH / (VOCAB - 1)
CONF = 1.0 - SMOOTH
import math
C_CONST = SMOOTH * math.log(FILL) + CONF * math.log(CONF)

ROWS_PER_BLOCK = 128


def _loss_kernel(pred_lo_ref, pred_hi_ref, tgt_ref, out_ref, acc_ref, cnt_ref,
                 *, num_blocks, half):
    i = pl.program_id(0)

    tgt = tgt_ref[...]
    lo = pred_lo_ref[...]
    hi = pred_hi_ref[...]
    col = jax.lax.broadcasted_iota(jnp.int32, lo.shape, 1)
    w_lo = jnp.where(col == tgt, CONF, FILL)
    w_hi = jnp.where(col + half == tgt, CONF, FILL)
    wsum = (jnp.sum(w_lo * lo, axis=1, keepdims=True)
            + jnp.sum(w_hi * hi, axis=1, keepdims=True))
    mask = (tgt != 0).astype(jnp.float32)
    block_loss = jnp.sum(mask * (C_CONST - wsum))
    block_cnt = jnp.sum(mask)

    @pl.when(i == 0)
    def _():
        acc_ref[0, 0] = 0.0
        cnt_ref[0, 0] = 0.0

    acc_ref[0, 0] += block_loss
    cnt_ref[0, 0] += block_cnt

    @pl.when(i == num_blocks - 1)
    def _():
        out_ref[...] = jnp.full((1, 1), acc_ref[0, 0] / cnt_ref[0, 0],
                                dtype=jnp.float32)


def kernel(prediction, target):
    _, n_tok, vocab = prediction.shape
    pred2d = prediction.reshape(n_tok, vocab)
    tgt_col = target.reshape(n_tok, 1)
    num_blocks = n_tok // ROWS_PER_BLOCK

    half = vocab // 2
    out = pl.pallas_call(
        functools.partial(_loss_kernel, num_blocks=num_blocks, half=half),
        grid=(num_blocks,),
        in_specs=[
            pl.BlockSpec((ROWS_PER_BLOCK, half), lambda i: (i, 0)),
            pl.BlockSpec((ROWS_PER_BLOCK, half), lambda i: (i, 1)),
            pl.BlockSpec((ROWS_PER_BLOCK, 1), lambda i: (i, 0)),
        ],
        out_specs=pl.BlockSpec((1, 1), lambda i: (0, 0)),
        out_shape=jax.ShapeDtypeStruct((1, 1), jnp.float32),
        scratch_shapes=[
            pltpu.SMEM((1, 1), jnp.float32),
            pltpu.SMEM((1, 1), jnp.float32),
        ],
    )(pred2d, pred2d, tgt_col)
    return out[0, 0]
